# Initial kernel scaffold; baseline (speedup 1.0000x reference)
import jax
import jax.numpy as jnp
from jax.experimental import pallas as pl


def kernel(h, x, edges, node_attr, We0, ge, be, We1, be1, Wm, bm, Wh0, bh0, gh, bh, Wh1, bh1, Wx0, bx0, Wx1):
    raise NotImplementedError("write your pallas kernel here")


# speedup vs baseline: 2.2147x; 2.2147x over previous
import functools

import jax
import jax.numpy as jnp
from jax import lax
from jax.experimental import pallas as pl
from jax.experimental.pallas import tpu as pltpu
from jax.experimental.pallas import tpu_sc as plsc

NC = 2
NS = 16
NW = NC * NS
L = 16
CHUNK = 80
BE = 512


def _mesh():
    return plsc.VectorSubcoreMesh(core_axis_name="c", subcore_axis_name="s",
                                  num_cores=NC, num_subcores=NS)


_SC_PARAMS = pltpu.CompilerParams(needs_layout_passes=False)


def _pq_tc(h, wat, wbt):
    n, _ = h.shape
    hdim = wat.shape[1]

    def body(h_ref, wa_ref, wb_ref, p_ref, q_ref):
        hv = h_ref[...]
        p_ref[...] = jnp.dot(hv, wa_ref[...], preferred_element_type=jnp.float32)
        q_ref[...] = jnp.dot(hv, wb_ref[...], preferred_element_type=jnp.float32)

    return pl.pallas_call(
        body,
        out_shape=[jax.ShapeDtypeStruct((n, hdim), jnp.float32)] * 2,
    )(h, wat, wbt)


def _gather_sc(p, q, xflat, iv, jv):
    n, hdim = p.shape
    e = iv.shape[0]
    epw = e // NW
    nchunks = epw // CHUNK
    assert epw % CHUNK == 0

    @functools.partial(
        pl.kernel,
        out_type=[
            jax.ShapeDtypeStruct((e, hdim), jnp.float32),
            jax.ShapeDtypeStruct((e, 8), jnp.float32),
        ],
        mesh=_mesh(),
        scratch_types=[
            pltpu.VMEM((n * 4,), jnp.float32),
            pltpu.VMEM((CHUNK,), jnp.int32),
            pltpu.VMEM((CHUNK,), jnp.int32),
            pltpu.VMEM((CHUNK, hdim), jnp.float32),
            pltpu.VMEM((CHUNK, hdim), jnp.float32),
            pltpu.VMEM((CHUNK, 8), jnp.float32),
            pltpu.SemaphoreType.DMA,
            pltpu.SemaphoreType.DMA,
        ],
        compiler_params=_SC_PARAMS,
    )
    def k(p_hbm, q_hbm, x_hbm, i_hbm, j_hbm, g_out, aux_out,
          xtab, ivb, jvb, bufp, bufq, bufa, semp, semq):
        wid = lax.axis_index("s") * NC + lax.axis_index("c")
        base0 = wid * epw
        pltpu.sync_copy(x_hbm, xtab)
        lanes = lax.iota(jnp.int32, L)

        def chunk_body(kk, _):
            base = base0 + kk * CHUNK
            pltpu.sync_copy(i_hbm.at[pl.ds(base, CHUNK)], ivb)
            pltpu.sync_copy(j_hbm.at[pl.ds(base, CHUNK)], jvb)
            cp = pltpu.async_copy(p_hbm.at[ivb], bufp, semp)
            cq = pltpu.async_copy(q_hbm.at[jvb], bufq, semq)
            for g in range(CHUNK // L):
                sl = pl.ds(g * L, L)
                ii = ivb[sl]
                jj = jvb[sl]
                xi = [plsc.load_gather(xtab, [ii * 4 + c]) for c in range(4)]
                xj = [plsc.load_gather(xtab, [jj * 4 + c]) for c in range(4)]
                xd = [a - b for a, b in zip(xi, xj)]
                ps = [v * v for v in xd]
                dq = [a * b for a, b in zip(xi, xj)]
                vals = [xd[0], xd[1], xd[2], xd[3],
                        ps[0] - ps[1] - ps[2] - ps[3],
                        dq[0] - dq[1] - dq[2] - dq[3]]
                rows = lanes + (g * L)
                for c, v in enumerate(vals):
                    colv = jnp.full((L,), c, jnp.int32)
                    plsc.store_scatter(bufa, [rows, colv], v)
            cp.wait()
            cq.wait()

            def add_row(r, _):
                for c in range(hdim // L):
                    s2 = pl.ds(c * L, L)
                    bufp[r, s2] = bufp[r, s2] + bufq[r, s2]
                return 0

            lax.fori_loop(0, CHUNK, add_row, 0)
            pltpu.sync_copy(bufp, g_out.at[pl.ds(base, CHUNK)])
            pltpu.sync_copy(bufa, aux_out.at[pl.ds(base, CHUNK)])
            return 0

        lax.fori_loop(0, nchunks, chunk_body, 0)

    return k(p, q, xflat, iv, jv)


def _psi(v):
    return jnp.sign(v) * jnp.log(jnp.abs(v) + 1.0)


def _stats_tc(g, aux, wn, wd):
    e, hdim = g.shape
    nb = e // BE

    def body(g_ref, a_ref, wn_ref, wd_ref, o_ref):
        @pl.when(pl.program_id(0) == 0)
        def _():
            o_ref[...] = jnp.zeros_like(o_ref)

        gv = g_ref[...]
        nr = a_ref[:, 4:5]
        dr = a_ref[:, 5:6]
        op = gv + _psi(nr) * wn_ref[...] + _psi(dr) * wd_ref[...]
        o_ref[0:1, :] += jnp.sum(op, axis=0, keepdims=True)
        o_ref[1:2, :] += jnp.sum(op * op, axis=0, keepdims=True)

    return pl.pallas_call(
        body,
        grid=(nb,),
        in_specs=[
            pl.BlockSpec((BE, hdim), lambda k: (k, 0)),
            pl.BlockSpec((BE, 8), lambda k: (k, 0)),
            pl.BlockSpec((1, hdim), lambda k: (0, 0)),
            pl.BlockSpec((1, hdim), lambda k: (0, 0)),
        ],
        out_specs=pl.BlockSpec((2, hdim), lambda k: (0, 0)),
        out_shape=jax.ShapeDtypeStruct((2, hdim), jnp.float32),
    )(g, aux, wn, wd)


def _mlp_tc(g, aux, wn, wd, scale, shift, we1t, be1, wm, bm, wx0t, bx0, wx1):
    e, hdim = g.shape
    nb = e // BE

    def body(g_ref, a_ref, wn_ref, wd_ref, sc_ref, sh_ref, w1_ref, b1_ref,
             wm_ref, bm_ref, wx0_ref, bx0_ref, wx1_ref, m_ref, p_ref):
        gv = g_ref[...]
        nr = a_ref[:, 4:5]
        dr = a_ref[:, 5:6]
        op = gv + _psi(nr) * wn_ref[...] + _psi(dr) * wd_ref[...]
        out = jnp.maximum(op * sc_ref[...] + sh_ref[...], 0.0)
        out = jnp.maximum(
            jnp.dot(out, w1_ref[...], preferred_element_type=jnp.float32)
            + b1_ref[...], 0.0)
        wg = jax.nn.sigmoid(
            jnp.sum(out * wm_ref[...], axis=1, keepdims=True) + bm_ref[...])
        m = out * wg
        m_ref[...] = m
        t1 = jnp.maximum(
            jnp.dot(m, wx0_ref[...], preferred_element_type=jnp.float32)
            + bx0_ref[...], 0.0)
        mx = jnp.sum(t1 * wx1_ref[...], axis=1, keepdims=True)
        cols = []
        for c in range(4):
            xc = a_ref[:, c:c + 1]
            cols.append(jnp.clip(xc * mx, -100.0, 100.0))
        one = jnp.ones_like(mx)
        zz = jnp.zeros((mx.shape[0], 123), jnp.float32)
        p_ref[...] = jnp.concatenate(cols + [one, zz], axis=1)

    blk0 = lambda k: (0, 0)
    return pl.pallas_call(
        body,
        grid=(nb,),
        in_specs=[
            pl.BlockSpec((BE, hdim), lambda k: (k, 0)),
            pl.BlockSpec((BE, 8), lambda k: (k, 0)),
            pl.BlockSpec((1, hdim), blk0),
            pl.BlockSpec((1, hdim), blk0),
            pl.BlockSpec((1, hdim), blk0),
            pl.BlockSpec((1, hdim), blk0),
            pl.BlockSpec((hdim, hdim), blk0),
            pl.BlockSpec((1, hdim), blk0),
            pl.BlockSpec((1, hdim), blk0),
            pl.BlockSpec((1, 1), blk0),
            pl.BlockSpec((hdim, hdim), blk0),
            pl.BlockSpec((1, hdim), blk0),
            pl.BlockSpec((1, hdim), blk0),
        ],
        out_specs=[
            pl.BlockSpec((BE, hdim), lambda k: (k, 0)),
            pl.BlockSpec((BE, 128), lambda k: (k, 0)),
        ],
        out_shape=[
            jax.ShapeDtypeStruct((e, hdim), jnp.float32),
            jax.ShapeDtypeStruct((e, 128), jnp.float32),
        ],
    )(g, aux, wn, wd, scale, shift, we1t, be1, wm, bm, wx0t, bx0, wx1)


def _scatter_one_sc(rows_arr, iv_sorted, perm, zrows):
    e, w = rows_arr.shape
    n = zrows.shape[0]
    epw = e // NW
    nchunks = epw // CHUNK
    r_main = (((n + NS - 1) // NS) + 7) // 8 * 8
    r_last = n - (NS - 1) * r_main
    assert r_last > 0 and r_last % 8 == 0

    @functools.partial(
        pl.kernel,
        out_type=jax.ShapeDtypeStruct((NC, n, w), jnp.float32),
        mesh=_mesh(),
        scratch_types=[
            pltpu.VMEM((CHUNK,), jnp.int32),
            pltpu.VMEM((CHUNK,), jnp.int32),
            pltpu.VMEM((CHUNK, w), jnp.float32),
            pltpu.VMEM_SHARED((n, w), jnp.float32),
        ],
        compiler_params=_SC_PARAMS,
    )
    def k(r_hbm, i_hbm, perm_hbm, z_hbm, acc_out, ivb, permb, rbuf, acc):
        cid = lax.axis_index("c")
        sid = lax.axis_index("s")
        wid = sid * NC + cid
        r0 = sid * r_main

        @pl.when(sid < NS - 1)
        def _():
            rows = pl.ds(r0, r_main)
            pltpu.sync_copy(z_hbm.at[rows], acc.at[rows])

        @pl.when(sid == NS - 1)
        def _():
            rows = pl.ds(r0, r_last)
            pltpu.sync_copy(z_hbm.at[rows], acc.at[rows])

        plsc.subcore_barrier()

        base0 = wid * epw

        def chunk_body(kk, _):
            base = base0 + kk * CHUNK
            pltpu.sync_copy(i_hbm.at[pl.ds(base, CHUNK)], ivb)
            pltpu.sync_copy(perm_hbm.at[pl.ds(base, CHUNK)], permb)
            pltpu.sync_copy(r_hbm.at[permb], rbuf)
            pltpu.sync_copy(rbuf, acc.at[ivb], add=True)
            return 0

        lax.fori_loop(0, nchunks, chunk_body, 0)
        plsc.subcore_barrier()

        @pl.when(sid < NS - 1)
        def _():
            rows = pl.ds(r0, r_main)
            pltpu.sync_copy(acc.at[rows], acc_out.at[cid, rows])

        @pl.when(sid == NS - 1)
        def _():
            rows = pl.ds(r0, r_last)
            pltpu.sync_copy(acc.at[rows], acc_out.at[cid, rows])

    return k(rows_arr, iv_sorted, perm, zrows)


def _scatter_sc(m, payload, iv_sorted, perm, zm, zp):
    accm = _scatter_one_sc(m, iv_sorted, perm, zm)
    accp = _scatter_one_sc(payload, iv_sorted, perm, zp)
    return accm, accp


def _node_tc(h, x, node_attr, accm, accp, w0h, w0m, w0a, bh0, gh, bh, wh1t, bh1):
    n, d = h.shape

    def body(h_ref, x_ref, na_ref, am_ref, ap_ref, w0h_ref, w0m_ref, w0a_ref,
             b0_ref, g_ref, b_ref, w1_ref, b1_ref, hn_ref, xn_ref):
        agm = am_ref[0] + am_ref[1]
        ap = ap_ref[0] + ap_ref[1]
        aggs = ap[:, 0:4]
        cnt = ap[:, 4:5]
        xn_ref[...] = x_ref[...] + aggs / jnp.maximum(cnt, 1.0)
        hv = h_ref[...]
        y = (jnp.dot(hv, w0h_ref[...], preferred_element_type=jnp.float32)
             + jnp.dot(agm, w0m_ref[...], preferred_element_type=jnp.float32)
             + jnp.dot(na_ref[...], w0a_ref[...], preferred_element_type=jnp.float32)
             + b0_ref[...])
        mu = jnp.mean(y, axis=0, keepdims=True)
        var = jnp.mean(y * y, axis=0, keepdims=True) - mu * mu
        hh = jnp.maximum(
            (y - mu) * lax.rsqrt(var + 1e-5) * g_ref[...] + b_ref[...], 0.0)
        hn_ref[...] = hv + jnp.dot(
            hh, w1_ref[...], preferred_element_type=jnp.float32) + b1_ref[...]

    return pl.pallas_call(
        body,
        out_shape=[
            jax.ShapeDtypeStruct((n, d), jnp.float32),
            jax.ShapeDtypeStruct((n, 4), jnp.float32),
        ],
    )(h, x, node_attr, accm, accp, w0h, w0m, w0a, bh0, gh, bh, wh1t, bh1)


def kernel(h, x, edges, node_attr, We0, ge, be, We1, be1, Wm, bm,
           Wh0, bh0, gh, bh, Wh1, bh1, Wx0, bx0, Wx1):
    n, d = h.shape
    e = edges.shape[1]
    hdim = We0.shape[0]
    iv = edges[0]
    jv = edges[1]

    wat = We0[:, :d].T
    wbt = We0[:, d:2 * d].T
    wn = We0[:, 2 * d][None, :]
    wd = We0[:, 2 * d + 1][None, :]

    p, q = _pq_tc(h, wat, wbt)
    g, aux = _gather_sc(p, q, x.reshape(-1), iv, jv)
    sums = _stats_tc(g, aux, wn, wd)
    mu = sums[0:1, :] / e
    var = sums[1:2, :] / e - mu * mu
    scale = ge[None, :] / jnp.sqrt(var + 1e-5)
    shift = be[None, :] - mu * scale

    m_arr, payload = _mlp_tc(
        g, aux, wn, wd, scale, shift, We1.T, be1[None, :], Wm,
        bm.reshape(1, 1), Wx0.T, bx0[None, :], Wx1)

    zm = jnp.zeros((n, hdim), jnp.float32)
    zp = jnp.zeros((n, 128), jnp.float32)
    perm = jnp.argsort(iv).astype(jnp.int32)
    iv_sorted = jnp.take(iv, perm)
    accm, accp = _scatter_sc(m_arr, payload, iv_sorted, perm, zm, zp)

    h_new, x_new = _node_tc(
        h, x, node_attr, accm, accp,
        Wh0[:, :d].T, Wh0[:, d:d + hdim].T, Wh0[:, d + hdim:].T,
        bh0[None, :], gh[None, :], bh[None, :], Wh1.T, bh1[None, :])
    return (h_new, x_new, m_arr)
